# baseline (speedup 1.0000x reference)
import functools

import jax
import jax.numpy as jnp
from jax.experimental import pallas as pl
from jax.experimental.pallas import tpu as pltpu

_B, _C_IN, _C_OUT, _N, _K = 4, 128, 128, 2048, 16
_TN = 512


def _pt_layer_kernel(x_ref, wqk_ref, wv_ref, wog_ref, bqk_ref, bv_ref,
                     bog_ref, out_ref):
    xb = x_ref[0]
    s = jnp.dot(wqk_ref[...], xb, preferred_element_type=jnp.float32)
    s = s + bqk_ref[...]
    m = jnp.max(s, axis=0, keepdims=True)
    e = jnp.exp(s - m)
    attn = e / jnp.sum(e, axis=0, keepdims=True)
    v = jnp.dot(wv_ref[...], xb, preferred_element_type=jnp.float32)
    v = v + bv_ref[...]
    xa = (float(_K) * attn) * v
    out = jnp.dot(wog_ref[...], xa, preferred_element_type=jnp.float32)
    out_ref[0] = out + bog_ref[...]


@functools.partial(jax.jit, static_argnames=())
def kernel(x, pos, Wq, bq, Wk, bk, Wv, bv, Wg, bg, Wo, bo):
    del pos
    B, C_in, N = x.shape
    C_out = Wq.shape[0]

    wqk = Wq - Wk
    bqk = (bq - bk)[:, None]
    wog = Wo + Wo @ Wg
    bog = (Wo @ bg + bo)[:, None]
    bv2 = bv[:, None]

    tn = _TN if N % _TN == 0 else N
    grid = (B, N // tn)

    wspec = pl.BlockSpec((C_out, C_in), lambda b, j: (0, 0))
    bspec = pl.BlockSpec((C_out, 1), lambda b, j: (0, 0))

    out = pl.pallas_call(
        _pt_layer_kernel,
        grid=grid,
        in_specs=[
            pl.BlockSpec((1, C_in, tn), lambda b, j: (b, 0, j)),
            wspec, wspec, wspec, bspec, bspec, bspec,
        ],
        out_specs=pl.BlockSpec((1, C_out, tn), lambda b, j: (b, 0, j)),
        out_shape=jax.ShapeDtypeStruct((B, C_out, N), jnp.float32),
        compiler_params=pltpu.CompilerParams(
            dimension_semantics=("parallel", "parallel")),
    )(x, wqk, Wv, wog, bqk, bv2, bog)
    return out


# speedup vs baseline: 1.3466x; 1.3466x over previous
import jax
import jax.numpy as jnp
from jax.experimental import pallas as pl
from jax.experimental.pallas import tpu as pltpu

_K = 16
_TN = 1024


def _pt_layer_kernel(x_ref, wq_ref, wk_ref, wv_ref, wg_ref, wo_ref,
                     bq_ref, bk_ref, bv_ref, bg_ref, bo_ref, out_ref):
    xb = x_ref[0]
    wqk = wq_ref[...] - wk_ref[...]
    s = jnp.dot(wqk, xb, preferred_element_type=jnp.float32)
    s = s + (bq_ref[...] - bk_ref[...])
    m = jnp.max(s, axis=0, keepdims=True)
    e = jnp.exp(s - m)
    attn = e / jnp.sum(e, axis=0, keepdims=True)
    v = jnp.dot(wv_ref[...], xb, preferred_element_type=jnp.float32)
    v = v + bv_ref[...]
    xa = (float(_K) * attn) * v
    wo = wo_ref[...]
    wog = wo + jnp.dot(wo, wg_ref[...], preferred_element_type=jnp.float32)
    bog = jnp.dot(wo, bg_ref[...], preferred_element_type=jnp.float32)
    bog = bog + bo_ref[...]
    out = jnp.dot(wog, xa, preferred_element_type=jnp.float32)
    out_ref[0] = out + bog


@jax.jit
def kernel(x, pos, Wq, bq, Wk, bk, Wv, bv, Wg, bg, Wo, bo):
    del pos
    B, C_in, N = x.shape
    C_out = Wq.shape[0]

    tn = _TN if N % _TN == 0 else N
    grid = (B, N // tn)

    wspec = pl.BlockSpec((C_out, C_in), lambda b, j: (0, 0))
    bspec = pl.BlockSpec((C_out, 1), lambda b, j: (0, 0))

    out = pl.pallas_call(
        _pt_layer_kernel,
        grid=grid,
        in_specs=[
            pl.BlockSpec((1, C_in, tn), lambda b, j: (b, 0, j)),
            wspec, wspec, wspec, wspec, wspec,
            bspec, bspec, bspec, bspec, bspec,
        ],
        out_specs=pl.BlockSpec((1, C_out, tn), lambda b, j: (b, 0, j)),
        out_shape=jax.ShapeDtypeStruct((B, C_out, N), jnp.float32),
        compiler_params=pltpu.CompilerParams(
            dimension_semantics=("parallel", "parallel")),
    )(x, Wq, Wk, Wv, Wg, Wo,
      bq[:, None], bk[:, None], bv[:, None], bg[:, None], bo[:, None])
    return out


# speedup vs baseline: 1.5706x; 1.1663x over previous
import jax
import jax.numpy as jnp
from jax.experimental import pallas as pl
from jax.experimental.pallas import tpu as pltpu

_K = 16
_TN = 2048


def _pt_layer_kernel(x_ref, wq_ref, wk_ref, wv_ref, wg_ref, wo_ref,
                     bq_ref, bk_ref, bv_ref, bg_ref, bo_ref, out_ref):
    xb = x_ref[0]
    wqk = wq_ref[...] - wk_ref[...]
    s = jnp.dot(wqk, xb, preferred_element_type=jnp.float32)
    s = s + (bq_ref[...] - bk_ref[...])
    m = jnp.max(s, axis=0, keepdims=True)
    e = jnp.exp(s - m)
    attn = e / jnp.sum(e, axis=0, keepdims=True)
    v = jnp.dot(wv_ref[...], xb, preferred_element_type=jnp.float32)
    v = v + bv_ref[...]
    xa = (float(_K) * attn) * v
    wo = wo_ref[...]
    wog = wo + jnp.dot(wo, wg_ref[...], preferred_element_type=jnp.float32)
    bog = jnp.dot(wo, bg_ref[...], preferred_element_type=jnp.float32)
    bog = bog + bo_ref[...]
    out = jnp.dot(wog, xa, preferred_element_type=jnp.float32)
    out_ref[0] = out + bog


@jax.jit
def kernel(x, pos, Wq, bq, Wk, bk, Wv, bv, Wg, bg, Wo, bo):
    del pos
    B, C_in, N = x.shape
    C_out = Wq.shape[0]

    tn = _TN if N % _TN == 0 else N
    grid = (B, N // tn)

    wspec = pl.BlockSpec((C_out, C_in), lambda b, j: (0, 0))
    bspec = pl.BlockSpec((C_out, 1), lambda b, j: (0, 0))

    out = pl.pallas_call(
        _pt_layer_kernel,
        grid=grid,
        in_specs=[
            pl.BlockSpec((1, C_in, tn), lambda b, j: (b, 0, j)),
            wspec, wspec, wspec, wspec, wspec,
            bspec, bspec, bspec, bspec, bspec,
        ],
        out_specs=pl.BlockSpec((1, C_out, tn), lambda b, j: (b, 0, j)),
        out_shape=jax.ShapeDtypeStruct((B, C_out, N), jnp.float32),
        compiler_params=pltpu.CompilerParams(
            dimension_semantics=("parallel", "parallel")),
    )(x, Wq, Wk, Wv, Wg, Wo,
      bq[:, None], bk[:, None], bv[:, None], bg[:, None], bo[:, None])
    return out


# speedup vs baseline: 1.6798x; 1.0696x over previous
import jax
import jax.numpy as jnp
from jax.experimental import pallas as pl
from jax.experimental.pallas import tpu as pltpu

_K = 16
_BB = 2


def _pt_layer_kernel(x_ref, wq_ref, wk_ref, wv_ref, wg_ref, wo_ref,
                     bq_ref, bk_ref, bv_ref, bg_ref, bo_ref, out_ref):
    wqk = wq_ref[...] - wk_ref[...]
    bqk = bq_ref[...] - bk_ref[...]
    wo = wo_ref[...]
    wog = wo + jnp.dot(wo, wg_ref[...], preferred_element_type=jnp.float32)
    bog = jnp.dot(wo, bg_ref[...], preferred_element_type=jnp.float32)
    bog = bog + bo_ref[...]
    for i in range(x_ref.shape[0]):
        xb = x_ref[i]
        s = jnp.dot(wqk, xb, preferred_element_type=jnp.float32) + bqk
        m = jnp.max(s, axis=0, keepdims=True)
        e = jnp.exp(s - m)
        attn = e / jnp.sum(e, axis=0, keepdims=True)
        v = jnp.dot(wv_ref[...], xb, preferred_element_type=jnp.float32)
        v = v + bv_ref[...]
        xa = (float(_K) * attn) * v
        out = jnp.dot(wog, xa, preferred_element_type=jnp.float32)
        out_ref[i] = out + bog


@jax.jit
def kernel(x, pos, Wq, bq, Wk, bk, Wv, bv, Wg, bg, Wo, bo):
    del pos
    B, C_in, N = x.shape
    C_out = Wq.shape[0]

    bb = _BB if B % _BB == 0 else B
    grid = (B // bb,)

    wspec = pl.BlockSpec((C_out, C_in), lambda b: (0, 0))
    bspec = pl.BlockSpec((C_out, 1), lambda b: (0, 0))

    out = pl.pallas_call(
        _pt_layer_kernel,
        grid=grid,
        in_specs=[
            pl.BlockSpec((bb, C_in, N), lambda b: (b, 0, 0)),
            wspec, wspec, wspec, wspec, wspec,
            bspec, bspec, bspec, bspec, bspec,
        ],
        out_specs=pl.BlockSpec((bb, C_out, N), lambda b: (b, 0, 0)),
        out_shape=jax.ShapeDtypeStruct((B, C_out, N), jnp.float32),
        compiler_params=pltpu.CompilerParams(
            dimension_semantics=("parallel",)),
    )(x, Wq, Wk, Wv, Wg, Wo,
      bq[:, None], bk[:, None], bv[:, None], bg[:, None], bo[:, None])
    return out
